# Initial kernel scaffold; baseline (speedup 1.0000x reference)
import functools

import jax
import jax.numpy as jnp
from jax import lax
from jax.experimental import pallas as pl
from jax.experimental.pallas import tpu as pltpu
from jax.experimental.pallas import tpu_sc as plsc

_N = 100000
_E = 3200000
_NP = 100352
_NPT = _NP // 16
_CH = 128
_NCHUNK = _E // _CH
_BLK = 1024
_NBLK = _NP // _BLK


def _deg_pass(col2, ew2):
    nchunk = col2.shape[0]
    per_tile = nchunk // 32 + 1
    mesh = plsc.VectorSubcoreMesh(core_axis_name="c", subcore_axis_name="s")

    @functools.partial(
        pl.kernel,
        out_type=jax.ShapeDtypeStruct((2 * _NP,), jnp.float32),
        mesh=mesh,
        scratch_types=[
            pltpu.VMEM_SHARED((_NP,), jnp.float32),
            pltpu.VMEM((_CH,), jnp.int32),
            pltpu.VMEM((_CH,), jnp.float32),
            pltpu.VMEM((_CH,), jnp.float32),
        ],
    )
    def k(col_hbm, ew_hbm, out_hbm, acc, col_b, ew_b, zbuf):
        c = lax.axis_index("c")
        s = lax.axis_index("s")
        wid = c * 16 + s

        @pl.loop(0, _CH, step=16)
        def _(i):
            zbuf[pl.ds(i, 16)] = jnp.zeros((16,), jnp.float32)

        @pl.loop(0, _NPT, step=_CH)
        def _(i):
            pltpu.sync_copy(zbuf, acc.at[pl.ds(s * _NPT + i, _CH)])

        plsc.subcore_barrier()

        @pl.loop(0, per_tile)
        def _(j):
            jj = j * 32 + wid

            @pl.when(jj < nchunk)
            def _():
                pltpu.sync_copy(col_hbm.at[jj], col_b)
                pltpu.sync_copy(ew_hbm.at[jj], ew_b)
                pltpu.sync_copy(ew_b, acc.at[col_b], add=True)

        plsc.subcore_barrier()
        pltpu.sync_copy(
            acc.at[pl.ds(s * _NPT, _NPT)],
            out_hbm.at[pl.ds(c * _NP + s * _NPT, _NPT)],
        )

    return k(col2, ew2)


def _conv_pass(tbl, row2, col2, ew2):
    nchunk = row2.shape[0]
    per_tile = nchunk // 16 + 1
    mesh = plsc.VectorSubcoreMesh(core_axis_name="c", subcore_axis_name="s")

    @functools.partial(
        pl.kernel,
        out_type=jax.ShapeDtypeStruct((2 * _NP, 16), jnp.float32),
        mesh=mesh,
        scratch_types=[
            pltpu.VMEM_SHARED((_NP, 16), jnp.float32),
            pltpu.VMEM((_CH,), jnp.int32),
            pltpu.VMEM((_CH,), jnp.int32),
            pltpu.VMEM((_CH,), jnp.float32),
            pltpu.VMEM((_CH, 16), jnp.float32),
            pltpu.VMEM((_CH, 16), jnp.float32),
            pltpu.SemaphoreType.DMA,
        ],
    )
    def k(t_hbm, row_hbm, col_hbm, ew_hbm, out_hbm, acc, row_b, col_b, ew_b,
          msg, zbuf, sem):
        c = lax.axis_index("c")
        s = lax.axis_index("s")
        cn = c * _NP

        @pl.loop(0, _CH)
        def _(i):
            zbuf[i] = jnp.zeros((16,), jnp.float32)

        @pl.loop(0, _NPT, step=_CH)
        def _(i):
            pltpu.sync_copy(zbuf, acc.at[pl.ds(s * _NPT + i, _CH)])

        plsc.subcore_barrier()

        iota = lax.iota(jnp.int32, 16)

        @pl.loop(0, per_tile)
        def _(j):
            jj = j * 16 + s

            @pl.when(jj < nchunk)
            def _():
                pltpu.sync_copy(row_hbm.at[jj], row_b)
                pltpu.sync_copy(col_hbm.at[jj], col_b)
                pltpu.sync_copy(ew_hbm.at[jj], ew_b)

                @pl.loop(0, _CH, step=16)
                def _(g):
                    row_b[pl.ds(g, 16)] = row_b[pl.ds(g, 16)] + cn

                pltpu.async_copy(t_hbm.at[row_b], msg, sem).wait()

                for g in range(8):
                    ew_v = ew_b[pl.ds(g * 16, 16)]
                    e_idx = iota + (g * 16)
                    for d in range(16):
                        d_idx = jnp.full((16,), d, jnp.int32)
                        v = plsc.load_gather(msg, [e_idx, d_idx])
                        plsc.store_scatter(msg, [e_idx, d_idx], v * ew_v)

                pltpu.sync_copy(msg, acc.at[col_b], add=True)

        plsc.subcore_barrier()
        pltpu.sync_copy(
            acc.at[pl.ds(s * _NPT, _NPT)],
            out_hbm.at[pl.ds(cn + s * _NPT, _NPT)],
        )

    return k(tbl, row2, col2, ew2)


def _mm_kernel(x_ref, w_ref, o_ref):
    o_ref[...] = jnp.dot(x_ref[...], w_ref[...],
                         preferred_element_type=jnp.float32)


def _matmul(x, w):
    n, din = x.shape
    dout = w.shape[1]
    return pl.pallas_call(
        _mm_kernel,
        grid=(n // _BLK,),
        in_specs=[
            pl.BlockSpec((_BLK, din), lambda i: (i, 0)),
            pl.BlockSpec((din, dout), lambda i: (0, 0)),
        ],
        out_specs=pl.BlockSpec((_BLK, dout), lambda i: (i, 0)),
        out_shape=jax.ShapeDtypeStruct((n, dout), jnp.float32),
    )(x, w)


def _scale_kernel(dp_ref, xw_ref, dis_ref, t_ref):
    deg = dp_ref[0] + dp_ref[1] + 1.0
    dis = lax.rsqrt(deg)
    dis_ref[...] = dis
    xw = xw_ref[...]
    t = dis * xw
    t_ref[0] = t[:, :16]
    t_ref[1] = t[:, 16:]


def _scale(dp2, xw):
    return pl.pallas_call(
        _scale_kernel,
        grid=(_NBLK,),
        in_specs=[
            pl.BlockSpec((2, _BLK, 1), lambda i: (0, i, 0)),
            pl.BlockSpec((_BLK, 32), lambda i: (i, 0)),
        ],
        out_specs=[
            pl.BlockSpec((_BLK, 1), lambda i: (i, 0)),
            pl.BlockSpec((2, _BLK, 16), lambda i: (0, i, 0)),
        ],
        out_shape=[
            jax.ShapeDtypeStruct((_NP, 1), jnp.float32),
            jax.ShapeDtypeStruct((2, _NP, 16), jnp.float32),
        ],
    )(dp2, xw)


def _finish_kernel(s_ref, t_ref, dis_ref, b_ref, w_ref, t2_ref):
    dis = dis_ref[...]
    sv = jnp.concatenate([s_ref[0], s_ref[1]], axis=1)
    tv = jnp.concatenate([t_ref[0], t_ref[1]], axis=1)
    h = jax.nn.relu(dis * (sv + tv) + b_ref[0:1, :])
    xw = jnp.dot(h, w_ref[...], preferred_element_type=jnp.float32)
    t2 = dis * xw
    t2_ref[0] = t2[:, :16]
    t2_ref[1] = t2[:, 16:]


def _finish_next(s2, t2, dis, b, w):
    return pl.pallas_call(
        _finish_kernel,
        grid=(_NBLK,),
        in_specs=[
            pl.BlockSpec((2, _BLK, 16), lambda i: (0, i, 0)),
            pl.BlockSpec((2, _BLK, 16), lambda i: (0, i, 0)),
            pl.BlockSpec((_BLK, 1), lambda i: (i, 0)),
            pl.BlockSpec((8, 32), lambda i: (0, 0)),
            pl.BlockSpec((32, 32), lambda i: (0, 0)),
        ],
        out_specs=pl.BlockSpec((2, _BLK, 16), lambda i: (0, i, 0)),
        out_shape=jax.ShapeDtypeStruct((2, _NP, 16), jnp.float32),
    )(s2, t2, dis, b, w)


def _head_kernel(s_ref, t_ref, dis_ref, b_ref, w_ref, bfc_ref, o_ref):
    dis = dis_ref[...]
    sv = jnp.concatenate([s_ref[0], s_ref[1]], axis=1)
    tv = jnp.concatenate([t_ref[0], t_ref[1]], axis=1)
    h = jax.nn.relu(dis * (sv + tv) + b_ref[0:1, :])
    o_ref[...] = jnp.dot(h, w_ref[...],
                         preferred_element_type=jnp.float32) + bfc_ref[0, 0]


def _head(s2, t2, dis, b, wfc, bfc):
    return pl.pallas_call(
        _head_kernel,
        grid=(_NBLK,),
        in_specs=[
            pl.BlockSpec((2, _BLK, 16), lambda i: (0, i, 0)),
            pl.BlockSpec((2, _BLK, 16), lambda i: (0, i, 0)),
            pl.BlockSpec((_BLK, 1), lambda i: (i, 0)),
            pl.BlockSpec((8, 32), lambda i: (0, 0)),
            pl.BlockSpec((32, 1), lambda i: (0, 0)),
            pl.BlockSpec((8, 8), lambda i: (0, 0)),
        ],
        out_specs=pl.BlockSpec((_BLK, 1), lambda i: (i, 0)),
        out_shape=jax.ShapeDtypeStruct((_NP, 1), jnp.float32),
    )(s2, t2, dis, b, wfc, bfc)


def kernel(x, c, ei, ew, W1, b1, W2, b2, Wfc, bfc):
    del c
    row2 = ei[0].reshape(_NCHUNK, _CH)
    col2 = ei[1].reshape(_NCHUNK, _CH)
    ew2 = ew.reshape(_NCHUNK, _CH)
    xp = jnp.pad(x, ((0, _NP - _N), (0, 0)))
    b1b = jnp.broadcast_to(b1, (8, 32))
    b2b = jnp.broadcast_to(b2, (8, 32))
    bfcb = jnp.broadcast_to(bfc.reshape(1, 1), (8, 8))

    dp = _deg_pass(col2, ew2)
    xw1 = _matmul(xp, W1)
    dis, t1 = _scale(dp.reshape(2, _NP, 1), xw1)
    s1 = _conv_pass(t1.reshape(2 * _NP, 16), row2, col2, ew2)
    t2 = _finish_next(s1.reshape(2, _NP, 16), t1, dis, b1b, W2)
    s2 = _conv_pass(t2.reshape(2 * _NP, 16), row2, col2, ew2)
    out = _head(s2.reshape(2, _NP, 16), t2, dis, b2b, Wfc, bfcb)
    return out[:_N]


# speedup vs baseline: 7.0783x; 7.0783x over previous
import functools

import jax
import jax.numpy as jnp
from jax import lax
from jax.experimental import pallas as pl
from jax.experimental.pallas import tpu as pltpu
from jax.experimental.pallas import tpu_sc as plsc

_N = 100000
_E = 3200000
_NP = 100352
_NPT = _NP // 16
_CH = 128
_NCHUNK = _E // _CH
_BLK = 1024
_NBLK = _NP // _BLK

_SC_PARAMS = pltpu.CompilerParams(needs_layout_passes=False,
                                  use_tc_tiling_on_sc=False)


def _deg_pass(col2, ew2):
    nchunk = col2.shape[0]
    per_tile = nchunk // 32 + 1
    mesh = plsc.VectorSubcoreMesh(core_axis_name="c", subcore_axis_name="s")

    @functools.partial(
        pl.kernel,
        out_type=jax.ShapeDtypeStruct((2 * _NP,), jnp.float32),
        mesh=mesh,
        scratch_types=[
            pltpu.VMEM_SHARED((_NP,), jnp.float32),
            pltpu.VMEM((_CH,), jnp.int32),
            pltpu.VMEM((_CH,), jnp.float32),
            pltpu.VMEM((_CH,), jnp.float32),
        ],
        compiler_params=_SC_PARAMS,
    )
    def k(col_hbm, ew_hbm, out_hbm, acc, col_b, ew_b, zbuf):
        c = lax.axis_index("c")
        s = lax.axis_index("s")
        wid = c * 16 + s

        @pl.loop(0, _CH, step=16)
        def _(i):
            zbuf[pl.ds(i, 16)] = jnp.zeros((16,), jnp.float32)

        @pl.loop(0, _NPT, step=_CH)
        def _(i):
            pltpu.sync_copy(zbuf, acc.at[pl.ds(s * _NPT + i, _CH)])

        plsc.subcore_barrier()

        @pl.loop(0, per_tile)
        def _(j):
            jj = j * 32 + wid

            @pl.when(jj < nchunk)
            def _():
                pltpu.sync_copy(col_hbm.at[jj], col_b)
                pltpu.sync_copy(ew_hbm.at[jj], ew_b)
                pltpu.sync_copy(ew_b, acc.at[col_b], add=True)

        plsc.subcore_barrier()
        pltpu.sync_copy(
            acc.at[pl.ds(s * _NPT, _NPT)],
            out_hbm.at[pl.ds(c * _NP + s * _NPT, _NPT)],
        )

    return k(col2, ew2)


def _conv_pass(tbl, row2, col2, ew2):
    nchunk = row2.shape[0]
    per_tile = nchunk // 16 + 1
    mesh = plsc.VectorSubcoreMesh(core_axis_name="c", subcore_axis_name="s")

    @functools.partial(
        pl.kernel,
        out_type=jax.ShapeDtypeStruct((2 * _NP, 16), jnp.float32),
        mesh=mesh,
        scratch_types=[
            pltpu.VMEM_SHARED((_NP, 16), jnp.float32),
            pltpu.VMEM((_CH,), jnp.int32),
            pltpu.VMEM((_CH,), jnp.int32),
            pltpu.VMEM((_CH,), jnp.float32),
            pltpu.VMEM((_CH, 16), jnp.float32),
            pltpu.VMEM((_CH, 16), jnp.float32),
            pltpu.SemaphoreType.DMA,
        ],
        compiler_params=_SC_PARAMS,
    )
    def k(t_hbm, row_hbm, col_hbm, ew_hbm, out_hbm, acc, row_b, col_b, ew_b,
          msg, zbuf, sem):
        c = lax.axis_index("c")
        s = lax.axis_index("s")
        cn = c * _NP

        @pl.loop(0, _CH)
        def _(i):
            zbuf[i] = jnp.zeros((16,), jnp.float32)

        @pl.loop(0, _NPT, step=_CH)
        def _(i):
            pltpu.sync_copy(zbuf, acc.at[pl.ds(s * _NPT + i, _CH)])

        plsc.subcore_barrier()

        iota = lax.iota(jnp.int32, 16)

        @pl.loop(0, per_tile)
        def _(j):
            jj = j * 16 + s

            @pl.when(jj < nchunk)
            def _():
                pltpu.sync_copy(row_hbm.at[jj], row_b)
                pltpu.sync_copy(col_hbm.at[jj], col_b)
                pltpu.sync_copy(ew_hbm.at[jj], ew_b)

                @pl.loop(0, _CH, step=16)
                def _(g):
                    row_b[pl.ds(g, 16)] = row_b[pl.ds(g, 16)] + cn

                pltpu.async_copy(t_hbm.at[row_b], msg, sem).wait()

                for g in range(8):
                    ew_v = ew_b[pl.ds(g * 16, 16)]
                    e_idx = iota + (g * 16)
                    for d in range(16):
                        d_idx = jnp.full((16,), d, jnp.int32)
                        v = plsc.load_gather(msg, [e_idx, d_idx])
                        plsc.store_scatter(msg, [e_idx, d_idx], v * ew_v)

                pltpu.sync_copy(msg, acc.at[col_b], add=True)

        plsc.subcore_barrier()
        pltpu.sync_copy(
            acc.at[pl.ds(s * _NPT, _NPT)],
            out_hbm.at[pl.ds(cn + s * _NPT, _NPT)],
        )

    return k(tbl, row2, col2, ew2)


def _mm_kernel(x_ref, w_ref, o_ref):
    o_ref[...] = jnp.dot(x_ref[...], w_ref[...],
                         preferred_element_type=jnp.float32)


def _matmul(x, w):
    n, din = x.shape
    dout = w.shape[1]
    return pl.pallas_call(
        _mm_kernel,
        grid=(n // _BLK,),
        in_specs=[
            pl.BlockSpec((_BLK, din), lambda i: (i, 0)),
            pl.BlockSpec((din, dout), lambda i: (0, 0)),
        ],
        out_specs=pl.BlockSpec((_BLK, dout), lambda i: (i, 0)),
        out_shape=jax.ShapeDtypeStruct((n, dout), jnp.float32),
    )(x, w)


def _scale_kernel(dp_ref, xw_ref, dis_ref, t_ref):
    deg = dp_ref[0] + dp_ref[1] + 1.0
    dis = lax.rsqrt(deg)
    dis_ref[...] = dis
    xw = xw_ref[...]
    t = dis * xw
    t_ref[0] = t[:, :16]
    t_ref[1] = t[:, 16:]


def _scale(dp2, xw):
    return pl.pallas_call(
        _scale_kernel,
        grid=(_NBLK,),
        in_specs=[
            pl.BlockSpec((2, _BLK, 1), lambda i: (0, i, 0)),
            pl.BlockSpec((_BLK, 32), lambda i: (i, 0)),
        ],
        out_specs=[
            pl.BlockSpec((_BLK, 1), lambda i: (i, 0)),
            pl.BlockSpec((2, _BLK, 16), lambda i: (0, i, 0)),
        ],
        out_shape=[
            jax.ShapeDtypeStruct((_NP, 1), jnp.float32),
            jax.ShapeDtypeStruct((2, _NP, 16), jnp.float32),
        ],
    )(dp2, xw)


def _finish_kernel(s_ref, t_ref, dis_ref, b_ref, w_ref, t2_ref):
    dis = dis_ref[...]
    sv = jnp.concatenate([s_ref[0], s_ref[1]], axis=1)
    tv = jnp.concatenate([t_ref[0], t_ref[1]], axis=1)
    h = jax.nn.relu(dis * (sv + tv) + b_ref[0:1, :])
    xw = jnp.dot(h, w_ref[...], preferred_element_type=jnp.float32)
    t2 = dis * xw
    t2_ref[0] = t2[:, :16]
    t2_ref[1] = t2[:, 16:]


def _finish_next(s2, t2, dis, b, w):
    return pl.pallas_call(
        _finish_kernel,
        grid=(_NBLK,),
        in_specs=[
            pl.BlockSpec((2, _BLK, 16), lambda i: (0, i, 0)),
            pl.BlockSpec((2, _BLK, 16), lambda i: (0, i, 0)),
            pl.BlockSpec((_BLK, 1), lambda i: (i, 0)),
            pl.BlockSpec((8, 32), lambda i: (0, 0)),
            pl.BlockSpec((32, 32), lambda i: (0, 0)),
        ],
        out_specs=pl.BlockSpec((2, _BLK, 16), lambda i: (0, i, 0)),
        out_shape=jax.ShapeDtypeStruct((2, _NP, 16), jnp.float32),
    )(s2, t2, dis, b, w)


def _head_kernel(s_ref, t_ref, dis_ref, b_ref, w_ref, bfc_ref, o_ref):
    dis = dis_ref[...]
    sv = jnp.concatenate([s_ref[0], s_ref[1]], axis=1)
    tv = jnp.concatenate([t_ref[0], t_ref[1]], axis=1)
    h = jax.nn.relu(dis * (sv + tv) + b_ref[0:1, :])
    o_ref[...] = jnp.dot(h, w_ref[...],
                         preferred_element_type=jnp.float32) + bfc_ref[0, 0]


def _head(s2, t2, dis, b, wfc, bfc):
    return pl.pallas_call(
        _head_kernel,
        grid=(_NBLK,),
        in_specs=[
            pl.BlockSpec((2, _BLK, 16), lambda i: (0, i, 0)),
            pl.BlockSpec((2, _BLK, 16), lambda i: (0, i, 0)),
            pl.BlockSpec((_BLK, 1), lambda i: (i, 0)),
            pl.BlockSpec((8, 32), lambda i: (0, 0)),
            pl.BlockSpec((32, 1), lambda i: (0, 0)),
            pl.BlockSpec((8, 8), lambda i: (0, 0)),
        ],
        out_specs=pl.BlockSpec((_BLK, 1), lambda i: (i, 0)),
        out_shape=jax.ShapeDtypeStruct((_NP, 1), jnp.float32),
    )(s2, t2, dis, b, wfc, bfc)


def kernel(x, c, ei, ew, W1, b1, W2, b2, Wfc, bfc):
    del c
    row2 = ei[0].reshape(_NCHUNK, _CH)
    col2 = ei[1].reshape(_NCHUNK, _CH)
    ew2 = ew.reshape(_NCHUNK, _CH)
    xp = jnp.pad(x, ((0, _NP - _N), (0, 0)))
    b1b = jnp.broadcast_to(b1, (8, 32))
    b2b = jnp.broadcast_to(b2, (8, 32))
    bfcb = jnp.broadcast_to(bfc.reshape(1, 1), (8, 8))

    dp = _deg_pass(col2, ew2)
    xw1 = _matmul(xp, W1)
    dis, t1 = _scale(dp.reshape(2, _NP, 1), xw1)
    s1 = _conv_pass(t1.reshape(2 * _NP, 16), row2, col2, ew2)
    t2 = _finish_next(s1.reshape(2, _NP, 16), t1, dis, b1b, W2)
    s2 = _conv_pass(t2.reshape(2 * _NP, 16), row2, col2, ew2)
    out = _head(s2.reshape(2, _NP, 16), t2, dis, b2b, Wfc, bfcb)
    return out[:_N]


# speedup vs baseline: 19.0573x; 2.6924x over previous
import functools

import jax
import jax.numpy as jnp
from jax import lax
from jax.experimental import pallas as pl
from jax.experimental.pallas import tpu as pltpu
from jax.experimental.pallas import tpu_sc as plsc

_N = 100000
_E = 3200000
_NP = 100352
_NPT = _NP // 16
_CH = 128
_NCHUNK = 25600
_E_PAD = _NCHUNK * _CH
_CPT = _NCHUNK // 16
_SZ = 2
_NSC = _CPT // _SZ
_SZD = 8
_CPW = _NCHUNK // 32
_NSCD = _CPW // _SZD
_ESC = _SZ * _CH
_BLK = 1024
_NBLK = _NP // _BLK

_SC_PARAMS = pltpu.CompilerParams(needs_layout_passes=False,
                                  use_tc_tiling_on_sc=False)


def _deg_pass(col2, ew2):
    mesh = plsc.VectorSubcoreMesh(core_axis_name="c", subcore_axis_name="s")

    @functools.partial(
        pl.kernel,
        out_type=jax.ShapeDtypeStruct((2 * _NP,), jnp.float32),
        mesh=mesh,
        scratch_types=(
            [pltpu.VMEM_SHARED((_NP,), jnp.float32)]
            + [pltpu.VMEM((_SZD, _CH), jnp.int32) for _ in range(4)]
            + [pltpu.VMEM((_SZD, _CH), jnp.float32) for _ in range(4)]
            + [pltpu.VMEM((2048,), jnp.float32)]
            + [pltpu.SemaphoreType.DMA for _ in range(8)]
        ),
        compiler_params=_SC_PARAMS,
    )
    def k(col_hbm, ew_hbm, out_hbm, acc,
          col0, col1, col2_, col3, ew0, ew1, ew2_, ew3, zbuf,
          st0, st1, st2, st3, ss0, ss1, ss2, ss3):
        c = lax.axis_index("c")
        s = lax.axis_index("s")
        wid = c * 16 + s
        cols = (col0, col1, col2_, col3)
        ews = (ew0, ew1, ew2_, ew3)
        sst = (st0, st1, st2, st3)
        sss = (ss0, ss1, ss2, ss3)
        lo = wid * _CPW

        @pl.loop(0, 2048, step=16)
        def _(i):
            zbuf[pl.ds(i, 16)] = jnp.zeros((16,), jnp.float32)

        @pl.loop(0, 3)
        def _(i):
            pltpu.sync_copy(zbuf, acc.at[pl.ds(s * _NPT + i * 2048, 2048)])

        pltpu.sync_copy(zbuf.at[pl.ds(0, _NPT - 3 * 2048)],
                        acc.at[pl.ds(s * _NPT + 3 * 2048, _NPT - 3 * 2048)])
        plsc.subcore_barrier()

        def stage(b, n):
            base = lo + n * _SZD
            pltpu.async_copy(col_hbm.at[pl.ds(base, _SZD)], cols[b], sst[b])
            pltpu.async_copy(ew_hbm.at[pl.ds(base, _SZD)], ews[b], sst[b])

        def wait_stage(b):
            pltpu.make_async_copy(col_hbm.at[pl.ds(0, _SZD)], cols[b], sst[b]).wait()
            pltpu.make_async_copy(ew_hbm.at[pl.ds(0, _SZD)], ews[b], sst[b]).wait()

        def fire_scatters(b):
            for kk in range(_SZD):
                pltpu.async_copy(ews[b].at[kk], acc.at[cols[b].at[kk]],
                                 sss[b], add=True)

        def drain_scatters(b):
            pltpu.make_async_copy(ew_hbm.at[pl.ds(0, _SZD)], ews[b], sss[b]).wait()

        stage(0, 0)
        stage(1, 1)

        @pl.loop(0, _NSCD // 4)
        def _(q):
            for j in range(4):
                n = q * 4 + j
                bn = j
                b2 = (j + 2) % 4

                def prep(n=n, bn=bn, b2=b2):
                    drain_scatters(b2)

                if j < 2:
                    @pl.when(q > 0)
                    def _():
                        prep()
                else:
                    prep()

                def st(n=n, b2=b2):
                    stage(b2, n + 2)

                if j < 2:
                    st()
                else:
                    @pl.when(q < _NSCD // 4 - 1)
                    def _():
                        st()

                wait_stage(bn)
                fire_scatters(bn)

        drain_scatters((_NSCD - 2) % 4)
        drain_scatters((_NSCD - 1) % 4)
        plsc.subcore_barrier()
        pltpu.sync_copy(
            acc.at[pl.ds(s * _NPT, _NPT)],
            out_hbm.at[pl.ds(c * _NP + s * _NPT, _NPT)],
        )

    return k(col2, ew2)


def _conv_pass(tbl, row2, col2, ew2):
    mesh = plsc.VectorSubcoreMesh(core_axis_name="c", subcore_axis_name="s")

    @functools.partial(
        pl.kernel,
        out_type=jax.ShapeDtypeStruct((2 * _NP, 16), jnp.float32),
        mesh=mesh,
        scratch_types=(
            [pltpu.VMEM_SHARED((_NP, 16), jnp.float32)]
            + [pltpu.VMEM((_SZ, _CH), jnp.int32) for _ in range(8)]
            + [pltpu.VMEM((_SZ, _CH), jnp.float32) for _ in range(4)]
            + [pltpu.VMEM((_ESC, 16), jnp.float32) for _ in range(4)]
            + [pltpu.VMEM((_CH, 16), jnp.float32)]
            + [pltpu.SemaphoreType.DMA for _ in range(12)]
        ),
        compiler_params=_SC_PARAMS,
    )
    def k(t_hbm, row_hbm, col_hbm, ew_hbm, out_hbm, acc,
          row0, row1, row2_, row3, col0, col1, col2_, col3,
          ew0, ew1, ew2_, ew3, msg0, msg1, msg2, msg3, zbuf,
          st0, st1, st2, st3, sg0, sg1, sg2, sg3, ss0, ss1, ss2, ss3):
        c = lax.axis_index("c")
        s = lax.axis_index("s")
        cn = c * _NP
        rows = (row0, row1, row2_, row3)
        cols = (col0, col1, col2_, col3)
        ews = (ew0, ew1, ew2_, ew3)
        msgs = (msg0, msg1, msg2, msg3)
        sst = (st0, st1, st2, st3)
        ssg = (sg0, sg1, sg2, sg3)
        sss = (ss0, ss1, ss2, ss3)
        lo = s * _CPT

        @pl.loop(0, _CH)
        def _(i):
            zbuf[i] = jnp.zeros((16,), jnp.float32)

        @pl.loop(0, _NPT, step=_CH)
        def _(i):
            pltpu.sync_copy(zbuf, acc.at[pl.ds(s * _NPT + i, _CH)])

        plsc.subcore_barrier()

        iota = lax.iota(jnp.int32, 16)
        dconsts = [jnp.full((16,), d, jnp.int32) for d in range(16)]

        def stage(b, n):
            base = lo + n * _SZ
            pltpu.async_copy(row_hbm.at[pl.ds(base, _SZ)], rows[b], sst[b])
            pltpu.async_copy(col_hbm.at[pl.ds(base, _SZ)], cols[b], sst[b])
            pltpu.async_copy(ew_hbm.at[pl.ds(base, _SZ)], ews[b], sst[b])

        def wait_stage(b):
            pltpu.make_async_copy(row_hbm.at[pl.ds(0, _SZ)], rows[b], sst[b]).wait()
            pltpu.make_async_copy(col_hbm.at[pl.ds(0, _SZ)], cols[b], sst[b]).wait()
            pltpu.make_async_copy(ew_hbm.at[pl.ds(0, _SZ)], ews[b], sst[b]).wait()

        def adjust(b):
            rowX = rows[b]

            @pl.loop(0, _SZ)
            def _(kk):
                for i in range(8):
                    rowX[kk, pl.ds(i * 16, 16)] = rowX[kk, pl.ds(i * 16, 16)] + cn

        def fire_gathers(b):
            for kk in range(_SZ):
                pltpu.async_copy(t_hbm.at[rows[b].at[kk]],
                                 msgs[b].at[pl.ds(kk * _CH, _CH)], ssg[b])

        def wait_gathers(b):
            pltpu.make_async_copy(t_hbm.at[pl.ds(0, _ESC)], msgs[b], ssg[b]).wait()

        def multiply(b):
            ewX, msgX = ews[b], msgs[b]

            @pl.loop(0, _ESC // 16)
            def _(g):
                kk = g // 8
                off = (g - kk * 8) * 16
                ew_v = ewX[kk, pl.ds(off, 16)]
                e_idx = iota + g * 16
                for d in range(16):
                    v = plsc.load_gather(msgX, [e_idx, dconsts[d]])
                    plsc.store_scatter(msgX, [e_idx, dconsts[d]], v * ew_v)

        def fire_scatters(b):
            for kk in range(_SZ):
                pltpu.async_copy(msgs[b].at[pl.ds(kk * _CH, _CH)],
                                 acc.at[cols[b].at[kk]], sss[b], add=True)

        def drain_scatters(b):
            pltpu.make_async_copy(t_hbm.at[pl.ds(0, _ESC)], msgs[b], sss[b]).wait()

        stage(0, 0)
        stage(1, 1)
        wait_stage(0)
        adjust(0)
        fire_gathers(0)

        @pl.loop(0, _NSC // 4)
        def _(q):
            for j in range(4):
                n = q * 4 + j
                bn = j
                b1 = (j + 1) % 4
                b2 = (j + 2) % 4

                def drain2(b2=b2):
                    drain_scatters(b2)

                if j < 2:
                    @pl.when(q > 0)
                    def _():
                        drain2()
                else:
                    drain2()

                def st2(n=n, b2=b2):
                    stage(b2, n + 2)

                if j < 2:
                    st2()
                else:
                    @pl.when(q < _NSC // 4 - 1)
                    def _():
                        st2()

                def launch1(b1=b1):
                    wait_stage(b1)
                    adjust(b1)
                    fire_gathers(b1)

                if j < 3:
                    launch1()
                else:
                    @pl.when(q < _NSC // 4 - 1)
                    def _():
                        launch1()

                wait_gathers(bn)
                multiply(bn)
                fire_scatters(bn)

        drain_scatters((_NSC - 2) % 4)
        drain_scatters((_NSC - 1) % 4)
        plsc.subcore_barrier()
        pltpu.sync_copy(
            acc.at[pl.ds(s * _NPT, _NPT)],
            out_hbm.at[pl.ds(cn + s * _NPT, _NPT)],
        )

    return k(tbl, row2, col2, ew2)


def _mm_kernel(x_ref, w_ref, o_ref):
    o_ref[...] = jnp.dot(x_ref[...], w_ref[...],
                         preferred_element_type=jnp.float32)


def _matmul(x, w):
    n, din = x.shape
    dout = w.shape[1]
    return pl.pallas_call(
        _mm_kernel,
        grid=(n // _BLK,),
        in_specs=[
            pl.BlockSpec((_BLK, din), lambda i: (i, 0)),
            pl.BlockSpec((din, dout), lambda i: (0, 0)),
        ],
        out_specs=pl.BlockSpec((_BLK, dout), lambda i: (i, 0)),
        out_shape=jax.ShapeDtypeStruct((n, dout), jnp.float32),
    )(x, w)


def _scale_kernel(dp_ref, xw_ref, dis_ref, t_ref):
    deg = dp_ref[0] + dp_ref[1] + 1.0
    dis = lax.rsqrt(deg)
    dis_ref[...] = dis
    xw = xw_ref[...]
    t = dis * xw
    t_ref[0] = t[:, :16]
    t_ref[1] = t[:, 16:]


def _scale(dp2, xw):
    return pl.pallas_call(
        _scale_kernel,
        grid=(_NBLK,),
        in_specs=[
            pl.BlockSpec((2, _BLK, 1), lambda i: (0, i, 0)),
            pl.BlockSpec((_BLK, 32), lambda i: (i, 0)),
        ],
        out_specs=[
            pl.BlockSpec((_BLK, 1), lambda i: (i, 0)),
            pl.BlockSpec((2, _BLK, 16), lambda i: (0, i, 0)),
        ],
        out_shape=[
            jax.ShapeDtypeStruct((_NP, 1), jnp.float32),
            jax.ShapeDtypeStruct((2, _NP, 16), jnp.float32),
        ],
    )(dp2, xw)


def _finish_kernel(s_ref, t_ref, dis_ref, b_ref, w_ref, t2_ref):
    dis = dis_ref[...]
    sv = jnp.concatenate([s_ref[0], s_ref[1]], axis=1)
    tv = jnp.concatenate([t_ref[0], t_ref[1]], axis=1)
    h = jax.nn.relu(dis * (sv + tv) + b_ref[0:1, :])
    xw = jnp.dot(h, w_ref[...], preferred_element_type=jnp.float32)
    t2 = dis * xw
    t2_ref[0] = t2[:, :16]
    t2_ref[1] = t2[:, 16:]


def _finish_next(s2, t2, dis, b, w):
    return pl.pallas_call(
        _finish_kernel,
        grid=(_NBLK,),
        in_specs=[
            pl.BlockSpec((2, _BLK, 16), lambda i: (0, i, 0)),
            pl.BlockSpec((2, _BLK, 16), lambda i: (0, i, 0)),
            pl.BlockSpec((_BLK, 1), lambda i: (i, 0)),
            pl.BlockSpec((8, 32), lambda i: (0, 0)),
            pl.BlockSpec((32, 32), lambda i: (0, 0)),
        ],
        out_specs=pl.BlockSpec((2, _BLK, 16), lambda i: (0, i, 0)),
        out_shape=jax.ShapeDtypeStruct((2, _NP, 16), jnp.float32),
    )(s2, t2, dis, b, w)


def _head_kernel(s_ref, t_ref, dis_ref, b_ref, w_ref, bfc_ref, o_ref):
    dis = dis_ref[...]
    sv = jnp.concatenate([s_ref[0], s_ref[1]], axis=1)
    tv = jnp.concatenate([t_ref[0], t_ref[1]], axis=1)
    h = jax.nn.relu(dis * (sv + tv) + b_ref[0:1, :])
    o_ref[...] = jnp.dot(h, w_ref[...],
                         preferred_element_type=jnp.float32) + bfc_ref[0, 0]


def _head(s2, t2, dis, b, wfc, bfc):
    return pl.pallas_call(
        _head_kernel,
        grid=(_NBLK,),
        in_specs=[
            pl.BlockSpec((2, _BLK, 16), lambda i: (0, i, 0)),
            pl.BlockSpec((2, _BLK, 16), lambda i: (0, i, 0)),
            pl.BlockSpec((_BLK, 1), lambda i: (i, 0)),
            pl.BlockSpec((8, 32), lambda i: (0, 0)),
            pl.BlockSpec((32, 1), lambda i: (0, 0)),
            pl.BlockSpec((8, 8), lambda i: (0, 0)),
        ],
        out_specs=pl.BlockSpec((_BLK, 1), lambda i: (i, 0)),
        out_shape=jax.ShapeDtypeStruct((_NP, 1), jnp.float32),
    )(s2, t2, dis, b, wfc, bfc)


def kernel(x, c, ei, ew, W1, b1, W2, b2, Wfc, bfc):
    del c
    pad = _E_PAD - _E
    rowp = jnp.concatenate([ei[0], jnp.zeros((pad,), ei.dtype)]).reshape(_NCHUNK, _CH)
    colp = jnp.concatenate([ei[1], jnp.full((pad,), _N, ei.dtype)]).reshape(_NCHUNK, _CH)
    ewp = jnp.concatenate([ew, jnp.zeros((pad,), ew.dtype)]).reshape(_NCHUNK, _CH)
    xp = jnp.pad(x, ((0, _NP - _N), (0, 0)))
    b1b = jnp.broadcast_to(b1, (8, 32))
    b2b = jnp.broadcast_to(b2, (8, 32))
    bfcb = jnp.broadcast_to(bfc.reshape(1, 1), (8, 8))

    dp = _deg_pass(colp, ewp)
    xw1 = _matmul(xp, W1)
    dis, t1 = _scale(dp.reshape(2, _NP, 1), xw1)
    s1 = _conv_pass(t1.reshape(2 * _NP, 16), rowp, colp, ewp)
    t2 = _finish_next(s1.reshape(2, _NP, 16), t1, dis, b1b, W2)
    s2 = _conv_pass(t2.reshape(2 * _NP, 16), rowp, colp, ewp)
    out = _head(s2.reshape(2, _NP, 16), t2, dis, b2b, Wfc, bfcb)
    return out[:_N]


# speedup vs baseline: 32.4245x; 1.7014x over previous
import functools

import jax
import jax.numpy as jnp
from jax import lax
from jax.experimental import pallas as pl
from jax.experimental.pallas import tpu as pltpu
from jax.experimental.pallas import tpu_sc as plsc

_N = 100000
_E = 3200000
_NP = 100352
_NPT = _NP // 16
_CH = 128
_NCHUNK = 25600
_E_PAD = _NCHUNK * _CH
_CPT = _NCHUNK // 16
_SZ = 2
_NSC = _CPT // _SZ
_SZD = 8
_CPW = _NCHUNK // 32
_NSCD = _CPW // _SZD
_ESC = _SZ * _CH
_BLK = 1024
_NBLK = _NP // _BLK

_SC_PARAMS = pltpu.CompilerParams(needs_layout_passes=False,
                                  use_tc_tiling_on_sc=False)


def _deg_pass(col2, ew2):
    mesh = plsc.VectorSubcoreMesh(core_axis_name="c", subcore_axis_name="s")

    @functools.partial(
        pl.kernel,
        out_type=jax.ShapeDtypeStruct((2 * _NP,), jnp.float32),
        mesh=mesh,
        scratch_types=(
            [pltpu.VMEM_SHARED((_NP,), jnp.float32)]
            + [pltpu.VMEM((_SZD, _CH), jnp.int32) for _ in range(4)]
            + [pltpu.VMEM((_SZD, _CH), jnp.float32) for _ in range(4)]
            + [pltpu.VMEM((2048,), jnp.float32)]
            + [pltpu.SemaphoreType.DMA for _ in range(8)]
        ),
        compiler_params=_SC_PARAMS,
    )
    def k(col_hbm, ew_hbm, out_hbm, acc,
          col0, col1, col2_, col3, ew0, ew1, ew2_, ew3, zbuf,
          st0, st1, st2, st3, ss0, ss1, ss2, ss3):
        c = lax.axis_index("c")
        s = lax.axis_index("s")
        wid = c * 16 + s
        cols = (col0, col1, col2_, col3)
        ews = (ew0, ew1, ew2_, ew3)
        sst = (st0, st1, st2, st3)
        sss = (ss0, ss1, ss2, ss3)
        lo = wid * _CPW

        @pl.loop(0, 2048, step=16)
        def _(i):
            zbuf[pl.ds(i, 16)] = jnp.zeros((16,), jnp.float32)

        @pl.loop(0, 3)
        def _(i):
            pltpu.sync_copy(zbuf, acc.at[pl.ds(s * _NPT + i * 2048, 2048)])

        pltpu.sync_copy(zbuf.at[pl.ds(0, _NPT - 3 * 2048)],
                        acc.at[pl.ds(s * _NPT + 3 * 2048, _NPT - 3 * 2048)])
        plsc.subcore_barrier()

        def stage(b, n):
            base = lo + n * _SZD
            pltpu.async_copy(col_hbm.at[pl.ds(base, _SZD)], cols[b], sst[b])
            pltpu.async_copy(ew_hbm.at[pl.ds(base, _SZD)], ews[b], sst[b])

        def wait_stage(b):
            pltpu.make_async_copy(col_hbm.at[pl.ds(0, _SZD)], cols[b], sst[b]).wait()
            pltpu.make_async_copy(ew_hbm.at[pl.ds(0, _SZD)], ews[b], sst[b]).wait()

        def fire_scatters(b):
            for kk in range(_SZD):
                pltpu.async_copy(ews[b].at[kk], acc.at[cols[b].at[kk]],
                                 sss[b], add=True)

        def drain_scatters(b):
            pltpu.make_async_copy(ew_hbm.at[pl.ds(0, _SZD)], ews[b], sss[b]).wait()

        stage(0, 0)
        stage(1, 1)

        @pl.loop(0, _NSCD // 4)
        def _(q):
            for j in range(4):
                n = q * 4 + j
                bn = j
                b2 = (j + 2) % 4

                def prep(n=n, bn=bn, b2=b2):
                    drain_scatters(b2)

                if j < 2:
                    @pl.when(q > 0)
                    def _():
                        prep()
                else:
                    prep()

                def st(n=n, b2=b2):
                    stage(b2, n + 2)

                if j < 2:
                    st()
                else:
                    @pl.when(q < _NSCD // 4 - 1)
                    def _():
                        st()

                wait_stage(bn)
                fire_scatters(bn)

        drain_scatters((_NSCD - 2) % 4)
        drain_scatters((_NSCD - 1) % 4)
        plsc.subcore_barrier()
        pltpu.sync_copy(
            acc.at[pl.ds(s * _NPT, _NPT)],
            out_hbm.at[pl.ds(c * _NP + s * _NPT, _NPT)],
        )

    return k(col2, ew2)


def _conv_pass(tbl, row2, col2, ew2):
    mesh = plsc.VectorSubcoreMesh(core_axis_name="c", subcore_axis_name="s")

    @functools.partial(
        pl.kernel,
        out_type=jax.ShapeDtypeStruct((2 * _NP, 16), jnp.float32),
        mesh=mesh,
        scratch_types=(
            [pltpu.VMEM_SHARED((_NP, 16), jnp.float32)]
            + [pltpu.VMEM((_SZ, _CH), jnp.int32) for _ in range(8)]
            + [pltpu.VMEM((_SZ, _CH), jnp.float32) for _ in range(4)]
            + [pltpu.VMEM((_ESC, 16), jnp.float32) for _ in range(4)]
            + [pltpu.VMEM((_CH, 16), jnp.float32)]
            + [pltpu.SemaphoreType.DMA for _ in range(12)]
        ),
        compiler_params=_SC_PARAMS,
    )
    def k(t_hbm, row_hbm, col_hbm, ew_hbm, out_hbm, acc,
          row0, row1, row2_, row3, col0, col1, col2_, col3,
          ew0, ew1, ew2_, ew3, msg0, msg1, msg2, msg3, zbuf,
          st0, st1, st2, st3, sg0, sg1, sg2, sg3, ss0, ss1, ss2, ss3):
        c = lax.axis_index("c")
        s = lax.axis_index("s")
        cn = c * _NP
        rows = (row0, row1, row2_, row3)
        cols = (col0, col1, col2_, col3)
        ews = (ew0, ew1, ew2_, ew3)
        msgs = (msg0, msg1, msg2, msg3)
        sst = (st0, st1, st2, st3)
        ssg = (sg0, sg1, sg2, sg3)
        sss = (ss0, ss1, ss2, ss3)
        lo = s * _CPT

        @pl.loop(0, _CH)
        def _(i):
            zbuf[i] = jnp.zeros((16,), jnp.float32)

        @pl.loop(0, _NPT, step=_CH)
        def _(i):
            pltpu.sync_copy(zbuf, acc.at[pl.ds(s * _NPT + i, _CH)])

        plsc.subcore_barrier()

        iota = lax.iota(jnp.int32, 16)
        dconsts = [jnp.full((16,), d, jnp.int32) for d in range(16)]

        def stage(b, n):
            base = lo + n * _SZ
            pltpu.async_copy(row_hbm.at[pl.ds(base, _SZ)], rows[b], sst[b])
            pltpu.async_copy(col_hbm.at[pl.ds(base, _SZ)], cols[b], sst[b])
            pltpu.async_copy(ew_hbm.at[pl.ds(base, _SZ)], ews[b], sst[b])

        def wait_stage(b):
            pltpu.make_async_copy(row_hbm.at[pl.ds(0, _SZ)], rows[b], sst[b]).wait()
            pltpu.make_async_copy(col_hbm.at[pl.ds(0, _SZ)], cols[b], sst[b]).wait()
            pltpu.make_async_copy(ew_hbm.at[pl.ds(0, _SZ)], ews[b], sst[b]).wait()

        def adjust(b):
            rowX = rows[b]

            @pl.loop(0, _SZ)
            def _(kk):
                for i in range(8):
                    rowX[kk, pl.ds(i * 16, 16)] = rowX[kk, pl.ds(i * 16, 16)] + cn

        def fire_gathers(b):
            for kk in range(_SZ):
                pltpu.async_copy(t_hbm.at[rows[b].at[kk]],
                                 msgs[b].at[pl.ds(kk * _CH, _CH)], ssg[b])

        def wait_gathers(b):
            pltpu.make_async_copy(t_hbm.at[pl.ds(0, _ESC)], msgs[b], ssg[b]).wait()

        def multiply(b):
            ewX, msgX = ews[b], msgs[b]

            @pl.loop(0, _ESC, step=16)
            def _(e0):
                kk = e0 // _CH
                off = e0 - kk * _CH
                ew_v = ewX[kk, pl.ds(off, 16)]
                for l in range(16):
                    bc = lax.gather(
                        ew_v, dconsts[l][:, None],
                        lax.GatherDimensionNumbers(
                            offset_dims=(), collapsed_slice_dims=(0,),
                            start_index_map=(0,)),
                        slice_sizes=(1,),
                        mode=lax.GatherScatterMode.PROMISE_IN_BOUNDS)
                    msgX[e0 + l] = msgX[e0 + l] * bc

        def fire_scatters(b):
            for kk in range(_SZ):
                pltpu.async_copy(msgs[b].at[pl.ds(kk * _CH, _CH)],
                                 acc.at[cols[b].at[kk]], sss[b], add=True)

        def drain_scatters(b):
            pltpu.make_async_copy(t_hbm.at[pl.ds(0, _ESC)], msgs[b], sss[b]).wait()

        stage(0, 0)
        stage(1, 1)
        wait_stage(0)
        adjust(0)
        fire_gathers(0)

        @pl.loop(0, _NSC // 4)
        def _(q):
            for j in range(4):
                n = q * 4 + j
                bn = j
                b1 = (j + 1) % 4
                b2 = (j + 2) % 4

                def drain2(b2=b2):
                    drain_scatters(b2)

                if j < 2:
                    @pl.when(q > 0)
                    def _():
                        drain2()
                else:
                    drain2()

                def st2(n=n, b2=b2):
                    stage(b2, n + 2)

                if j < 2:
                    st2()
                else:
                    @pl.when(q < _NSC // 4 - 1)
                    def _():
                        st2()

                def launch1(b1=b1):
                    wait_stage(b1)
                    adjust(b1)
                    fire_gathers(b1)

                if j < 3:
                    launch1()
                else:
                    @pl.when(q < _NSC // 4 - 1)
                    def _():
                        launch1()

                wait_gathers(bn)
                multiply(bn)
                fire_scatters(bn)

        drain_scatters((_NSC - 2) % 4)
        drain_scatters((_NSC - 1) % 4)
        plsc.subcore_barrier()
        pltpu.sync_copy(
            acc.at[pl.ds(s * _NPT, _NPT)],
            out_hbm.at[pl.ds(cn + s * _NPT, _NPT)],
        )

    return k(tbl, row2, col2, ew2)


def _mm_kernel(x_ref, w_ref, o_ref):
    o_ref[...] = jnp.dot(x_ref[...], w_ref[...],
                         preferred_element_type=jnp.float32)


def _matmul(x, w):
    n, din = x.shape
    dout = w.shape[1]
    return pl.pallas_call(
        _mm_kernel,
        grid=(n // _BLK,),
        in_specs=[
            pl.BlockSpec((_BLK, din), lambda i: (i, 0)),
            pl.BlockSpec((din, dout), lambda i: (0, 0)),
        ],
        out_specs=pl.BlockSpec((_BLK, dout), lambda i: (i, 0)),
        out_shape=jax.ShapeDtypeStruct((n, dout), jnp.float32),
    )(x, w)


def _scale_kernel(dp_ref, xw_ref, dis_ref, t_ref):
    deg = dp_ref[0] + dp_ref[1] + 1.0
    dis = lax.rsqrt(deg)
    dis_ref[...] = dis
    xw = xw_ref[...]
    t = dis * xw
    t_ref[0] = t[:, :16]
    t_ref[1] = t[:, 16:]


def _scale(dp2, xw):
    return pl.pallas_call(
        _scale_kernel,
        grid=(_NBLK,),
        in_specs=[
            pl.BlockSpec((2, _BLK, 1), lambda i: (0, i, 0)),
            pl.BlockSpec((_BLK, 32), lambda i: (i, 0)),
        ],
        out_specs=[
            pl.BlockSpec((_BLK, 1), lambda i: (i, 0)),
            pl.BlockSpec((2, _BLK, 16), lambda i: (0, i, 0)),
        ],
        out_shape=[
            jax.ShapeDtypeStruct((_NP, 1), jnp.float32),
            jax.ShapeDtypeStruct((2, _NP, 16), jnp.float32),
        ],
    )(dp2, xw)


def _finish_kernel(s_ref, t_ref, dis_ref, b_ref, w_ref, t2_ref):
    dis = dis_ref[...]
    sv = jnp.concatenate([s_ref[0], s_ref[1]], axis=1)
    tv = jnp.concatenate([t_ref[0], t_ref[1]], axis=1)
    h = jax.nn.relu(dis * (sv + tv) + b_ref[0:1, :])
    xw = jnp.dot(h, w_ref[...], preferred_element_type=jnp.float32)
    t2 = dis * xw
    t2_ref[0] = t2[:, :16]
    t2_ref[1] = t2[:, 16:]


def _finish_next(s2, t2, dis, b, w):
    return pl.pallas_call(
        _finish_kernel,
        grid=(_NBLK,),
        in_specs=[
            pl.BlockSpec((2, _BLK, 16), lambda i: (0, i, 0)),
            pl.BlockSpec((2, _BLK, 16), lambda i: (0, i, 0)),
            pl.BlockSpec((_BLK, 1), lambda i: (i, 0)),
            pl.BlockSpec((8, 32), lambda i: (0, 0)),
            pl.BlockSpec((32, 32), lambda i: (0, 0)),
        ],
        out_specs=pl.BlockSpec((2, _BLK, 16), lambda i: (0, i, 0)),
        out_shape=jax.ShapeDtypeStruct((2, _NP, 16), jnp.float32),
    )(s2, t2, dis, b, w)


def _head_kernel(s_ref, t_ref, dis_ref, b_ref, w_ref, bfc_ref, o_ref):
    dis = dis_ref[...]
    sv = jnp.concatenate([s_ref[0], s_ref[1]], axis=1)
    tv = jnp.concatenate([t_ref[0], t_ref[1]], axis=1)
    h = jax.nn.relu(dis * (sv + tv) + b_ref[0:1, :])
    o_ref[...] = jnp.dot(h, w_ref[...],
                         preferred_element_type=jnp.float32) + bfc_ref[0, 0]


def _head(s2, t2, dis, b, wfc, bfc):
    return pl.pallas_call(
        _head_kernel,
        grid=(_NBLK,),
        in_specs=[
            pl.BlockSpec((2, _BLK, 16), lambda i: (0, i, 0)),
            pl.BlockSpec((2, _BLK, 16), lambda i: (0, i, 0)),
            pl.BlockSpec((_BLK, 1), lambda i: (i, 0)),
            pl.BlockSpec((8, 32), lambda i: (0, 0)),
            pl.BlockSpec((32, 1), lambda i: (0, 0)),
            pl.BlockSpec((8, 8), lambda i: (0, 0)),
        ],
        out_specs=pl.BlockSpec((_BLK, 1), lambda i: (i, 0)),
        out_shape=jax.ShapeDtypeStruct((_NP, 1), jnp.float32),
    )(s2, t2, dis, b, wfc, bfc)


def kernel(x, c, ei, ew, W1, b1, W2, b2, Wfc, bfc):
    del c
    pad = _E_PAD - _E
    rowp = jnp.concatenate([ei[0], jnp.zeros((pad,), ei.dtype)]).reshape(_NCHUNK, _CH)
    colp = jnp.concatenate([ei[1], jnp.full((pad,), _N, ei.dtype)]).reshape(_NCHUNK, _CH)
    ewp = jnp.concatenate([ew, jnp.zeros((pad,), ew.dtype)]).reshape(_NCHUNK, _CH)
    xp = jnp.pad(x, ((0, _NP - _N), (0, 0)))
    b1b = jnp.broadcast_to(b1, (8, 32))
    b2b = jnp.broadcast_to(b2, (8, 32))
    bfcb = jnp.broadcast_to(bfc.reshape(1, 1), (8, 8))

    dp = _deg_pass(colp, ewp)
    xw1 = _matmul(xp, W1)
    dis, t1 = _scale(dp.reshape(2, _NP, 1), xw1)
    s1 = _conv_pass(t1.reshape(2 * _NP, 16), rowp, colp, ewp)
    t2 = _finish_next(s1.reshape(2, _NP, 16), t1, dis, b1b, W2)
    s2 = _conv_pass(t2.reshape(2 * _NP, 16), rowp, colp, ewp)
    out = _head(s2.reshape(2, _NP, 16), t2, dis, b2b, Wfc, bfcb)
    return out[:_N]


# speedup vs baseline: 37.9603x; 1.1707x over previous
import functools

import jax
import jax.numpy as jnp
from jax import lax
from jax.experimental import pallas as pl
from jax.experimental.pallas import tpu as pltpu
from jax.experimental.pallas import tpu_sc as plsc

_N = 100000
_E = 3200000
_NP = 100352
_NPT = _NP // 16
_CH = 128
_NCHUNK = 25600
_E_PAD = _NCHUNK * _CH
_CPT = _NCHUNK // 16
_SZ = 2
_NSC = _CPT // _SZ
_SZD = 8
_CPW = _NCHUNK // 32
_NSCD = _CPW // _SZD
_ESC = _SZ * _CH
_BLK = 1024
_NBLK = _NP // _BLK

_SC_PARAMS = pltpu.CompilerParams(needs_layout_passes=False,
                                  use_tc_tiling_on_sc=False)


def _deg_pass(col2, ew2):
    mesh = plsc.VectorSubcoreMesh(core_axis_name="c", subcore_axis_name="s")

    @functools.partial(
        pl.kernel,
        out_type=jax.ShapeDtypeStruct((2 * _NP,), jnp.float32),
        mesh=mesh,
        scratch_types=(
            [pltpu.VMEM_SHARED((_NP,), jnp.float32)]
            + [pltpu.VMEM((_SZD, _CH), jnp.int32) for _ in range(4)]
            + [pltpu.VMEM((_SZD, _CH), jnp.float32) for _ in range(4)]
            + [pltpu.VMEM((2048,), jnp.float32)]
            + [pltpu.SemaphoreType.DMA for _ in range(8)]
        ),
        compiler_params=_SC_PARAMS,
    )
    def k(col_hbm, ew_hbm, out_hbm, acc,
          col0, col1, col2_, col3, ew0, ew1, ew2_, ew3, zbuf,
          st0, st1, st2, st3, ss0, ss1, ss2, ss3):
        c = lax.axis_index("c")
        s = lax.axis_index("s")
        wid = c * 16 + s
        cols = (col0, col1, col2_, col3)
        ews = (ew0, ew1, ew2_, ew3)
        sst = (st0, st1, st2, st3)
        sss = (ss0, ss1, ss2, ss3)
        lo = wid * _CPW

        @pl.loop(0, 2048, step=16)
        def _(i):
            zbuf[pl.ds(i, 16)] = jnp.zeros((16,), jnp.float32)

        @pl.loop(0, 3)
        def _(i):
            pltpu.sync_copy(zbuf, acc.at[pl.ds(s * _NPT + i * 2048, 2048)])

        pltpu.sync_copy(zbuf.at[pl.ds(0, _NPT - 3 * 2048)],
                        acc.at[pl.ds(s * _NPT + 3 * 2048, _NPT - 3 * 2048)])
        plsc.subcore_barrier()

        def stage(b, n):
            base = lo + n * _SZD
            pltpu.async_copy(col_hbm.at[pl.ds(base, _SZD)], cols[b], sst[b])
            pltpu.async_copy(ew_hbm.at[pl.ds(base, _SZD)], ews[b], sst[b])

        def wait_stage(b):
            pltpu.make_async_copy(col_hbm.at[pl.ds(0, _SZD)], cols[b], sst[b]).wait()
            pltpu.make_async_copy(ew_hbm.at[pl.ds(0, _SZD)], ews[b], sst[b]).wait()

        def fire_scatters(b):
            for kk in range(_SZD):
                pltpu.async_copy(ews[b].at[kk], acc.at[cols[b].at[kk]],
                                 sss[b], add=True)

        def drain_scatters(b):
            pltpu.make_async_copy(ew_hbm.at[pl.ds(0, _SZD)], ews[b], sss[b]).wait()

        stage(0, 0)
        stage(1, 1)

        @pl.loop(0, _NSCD // 4)
        def _(q):
            for j in range(4):
                n = q * 4 + j
                bn = j
                b2 = (j + 2) % 4

                def prep(n=n, bn=bn, b2=b2):
                    drain_scatters(b2)

                if j < 2:
                    @pl.when(q > 0)
                    def _():
                        prep()
                else:
                    prep()

                def st(n=n, b2=b2):
                    stage(b2, n + 2)

                if j < 2:
                    st()
                else:
                    @pl.when(q < _NSCD // 4 - 1)
                    def _():
                        st()

                wait_stage(bn)
                fire_scatters(bn)

        drain_scatters((_NSCD - 2) % 4)
        drain_scatters((_NSCD - 1) % 4)
        plsc.subcore_barrier()
        pltpu.sync_copy(
            acc.at[pl.ds(s * _NPT, _NPT)],
            out_hbm.at[pl.ds(c * _NP + s * _NPT, _NPT)],
        )

    return k(col2, ew2)


def _conv_pass(tbl, row2, col2, ew2):
    mesh = plsc.VectorSubcoreMesh(core_axis_name="c", subcore_axis_name="s")

    @functools.partial(
        pl.kernel,
        out_type=jax.ShapeDtypeStruct((2 * _NP, 16), jnp.float32),
        mesh=mesh,
        scratch_types=(
            [pltpu.VMEM_SHARED((_NP, 16), jnp.float32)]
            + [pltpu.VMEM((_SZ, _CH), jnp.int32) for _ in range(8)]
            + [pltpu.VMEM((_SZ, _CH), jnp.float32) for _ in range(4)]
            + [pltpu.VMEM((_ESC, 16), jnp.float32) for _ in range(4)]
            + [pltpu.VMEM((_CH, 16), jnp.float32)]
            + [pltpu.SemaphoreType.DMA for _ in range(12)]
        ),
        compiler_params=_SC_PARAMS,
    )
    def k(t_hbm, row_hbm, col_hbm, ew_hbm, out_hbm, acc,
          row0, row1, row2_, row3, col0, col1, col2_, col3,
          ew0, ew1, ew2_, ew3, msg0, msg1, msg2, msg3, zbuf,
          st0, st1, st2, st3, sg0, sg1, sg2, sg3, ss0, ss1, ss2, ss3):
        c = lax.axis_index("c")
        s = lax.axis_index("s")
        cn = c * _NP
        rows = (row0, row1, row2_, row3)
        cols = (col0, col1, col2_, col3)
        ews = (ew0, ew1, ew2_, ew3)
        msgs = (msg0, msg1, msg2, msg3)
        sst = (st0, st1, st2, st3)
        ssg = (sg0, sg1, sg2, sg3)
        sss = (ss0, ss1, ss2, ss3)
        lo = s * _CPT

        @pl.loop(0, _CH)
        def _(i):
            zbuf[i] = jnp.zeros((16,), jnp.float32)

        @pl.loop(0, _NPT, step=_CH)
        def _(i):
            pltpu.sync_copy(zbuf, acc.at[pl.ds(s * _NPT + i, _CH)])

        plsc.subcore_barrier()

        iota = lax.iota(jnp.int32, 16)
        dconsts = [jnp.full((16,), d, jnp.int32) for d in range(16)]

        def stage(b, n):
            base = lo + n * _SZ
            pltpu.async_copy(row_hbm.at[pl.ds(base, _SZ)], rows[b], sst[b])
            pltpu.async_copy(col_hbm.at[pl.ds(base, _SZ)], cols[b], sst[b])
            pltpu.async_copy(ew_hbm.at[pl.ds(base, _SZ)], ews[b], sst[b])

        def wait_stage(b):
            pltpu.make_async_copy(row_hbm.at[pl.ds(0, _SZ)], rows[b], sst[b]).wait()
            pltpu.make_async_copy(col_hbm.at[pl.ds(0, _SZ)], cols[b], sst[b]).wait()
            pltpu.make_async_copy(ew_hbm.at[pl.ds(0, _SZ)], ews[b], sst[b]).wait()

        def adjust(b):
            rowX = rows[b]

            @pl.loop(0, _SZ)
            def _(kk):
                for i in range(8):
                    rowX[kk, pl.ds(i * 16, 16)] = rowX[kk, pl.ds(i * 16, 16)] + cn

        def fire_gathers(b):
            for kk in range(_SZ):
                pltpu.async_copy(t_hbm.at[rows[b].at[kk]],
                                 msgs[b].at[pl.ds(kk * _CH, _CH)], ssg[b])

        def wait_gathers(b):
            pltpu.make_async_copy(t_hbm.at[pl.ds(0, _ESC)], msgs[b], ssg[b]).wait()

        def multiply(b):
            ewX, msgX = ews[b], msgs[b]

            @pl.loop(0, _ESC, step=16)
            def _(e0):
                kk = e0 // _CH
                off = e0 - kk * _CH
                ew_v = ewX[kk, pl.ds(off, 16)]
                for l in range(16):
                    bc = lax.gather(
                        ew_v, dconsts[l][:, None],
                        lax.GatherDimensionNumbers(
                            offset_dims=(), collapsed_slice_dims=(0,),
                            start_index_map=(0,)),
                        slice_sizes=(1,),
                        mode=lax.GatherScatterMode.PROMISE_IN_BOUNDS)
                    msgX[e0 + l] = msgX[e0 + l] * bc

        def fire_scatters(b):
            for kk in range(_SZ):
                pltpu.async_copy(msgs[b].at[pl.ds(kk * _CH, _CH)],
                                 acc.at[cols[b].at[kk]], sss[b], add=True)

        def drain_scatters(b):
            pltpu.make_async_copy(t_hbm.at[pl.ds(0, _ESC)], msgs[b], sss[b]).wait()

        stage(0, 0)
        stage(1, 1)
        wait_stage(0)
        adjust(0)
        fire_gathers(0)

        @pl.loop(0, _NSC // 4)
        def _(q):
            for j in range(4):
                n = q * 4 + j
                bn = j
                b1 = (j + 1) % 4
                b2 = (j + 2) % 4

                def drain2(b2=b2):
                    drain_scatters(b2)

                if j < 2:
                    @pl.when(q > 0)
                    def _():
                        drain2()
                else:
                    drain2()

                def st2(n=n, b2=b2):
                    stage(b2, n + 2)

                if j < 2:
                    st2()
                else:
                    @pl.when(q < _NSC // 4 - 1)
                    def _():
                        st2()

                def launch1(b1=b1):
                    wait_stage(b1)
                    adjust(b1)
                    fire_gathers(b1)

                if j < 3:
                    launch1()
                else:
                    @pl.when(q < _NSC // 4 - 1)
                    def _():
                        launch1()

                wait_gathers(bn)
                multiply(bn)
                fire_scatters(bn)

        drain_scatters((_NSC - 2) % 4)
        drain_scatters((_NSC - 1) % 4)
        plsc.subcore_barrier()
        pltpu.sync_copy(
            acc.at[pl.ds(s * _NPT, _NPT)],
            out_hbm.at[pl.ds(cn + s * _NPT, _NPT)],
        )

    return k(tbl, row2, col2, ew2)


_BLKL = 7168
_NBLKL = _NP // _BLKL


def _scale_t_kernel(dp_ref, xt_ref, w_ref, dis_ref, t_ref):
    deg = dp_ref[0:1, :] + dp_ref[1:2, :] + 1.0
    dis = lax.rsqrt(deg)
    dis_ref[...] = jnp.broadcast_to(dis, (8, _BLKL))
    xw = jnp.dot(w_ref[...], xt_ref[...], preferred_element_type=jnp.float32)
    t_ref[...] = dis * xw


def _scale_t(dp8, xT, w1t):
    return pl.pallas_call(
        _scale_t_kernel,
        grid=(_NBLKL,),
        in_specs=[
            pl.BlockSpec((8, _BLKL), lambda i: (0, i)),
            pl.BlockSpec((16, _BLKL), lambda i: (0, i)),
            pl.BlockSpec((32, 16), lambda i: (0, 0)),
        ],
        out_specs=[
            pl.BlockSpec((8, _BLKL), lambda i: (0, i)),
            pl.BlockSpec((32, _BLKL), lambda i: (0, i)),
        ],
        out_shape=[
            jax.ShapeDtypeStruct((8, _NP), jnp.float32),
            jax.ShapeDtypeStruct((32, _NP), jnp.float32),
        ],
    )(dp8, xT, w1t)


def _finish_t_kernel(s_ref, t_ref, dis_ref, b_ref, w_ref, t2_ref):
    dis = dis_ref[0:1, :]
    h = jax.nn.relu(dis * (s_ref[...] + t_ref[...]) + b_ref[...])
    t2_ref[...] = dis * jnp.dot(w_ref[...], h,
                                preferred_element_type=jnp.float32)


def _finish_t(sT, tT, dis, bc, w2t):
    return pl.pallas_call(
        _finish_t_kernel,
        grid=(_NBLKL,),
        in_specs=[
            pl.BlockSpec((32, _BLKL), lambda i: (0, i)),
            pl.BlockSpec((32, _BLKL), lambda i: (0, i)),
            pl.BlockSpec((8, _BLKL), lambda i: (0, i)),
            pl.BlockSpec((32, 1), lambda i: (0, 0)),
            pl.BlockSpec((32, 32), lambda i: (0, 0)),
        ],
        out_specs=pl.BlockSpec((32, _BLKL), lambda i: (0, i)),
        out_shape=jax.ShapeDtypeStruct((32, _NP), jnp.float32),
    )(sT, tT, dis, bc, w2t)


def _head_t_kernel(s_ref, t_ref, dis_ref, b_ref, w_ref, bfc_ref, o_ref):
    dis = dis_ref[0:1, :]
    h = jax.nn.relu(dis * (s_ref[...] + t_ref[...]) + b_ref[...])
    o_ref[...] = (jnp.dot(w_ref[...], h, preferred_element_type=jnp.float32)
                  + bfc_ref[0, 0])


def _head_t(sT, tT, dis, bc, wfct8, bfc):
    return pl.pallas_call(
        _head_t_kernel,
        grid=(_NBLKL,),
        in_specs=[
            pl.BlockSpec((32, _BLKL), lambda i: (0, i)),
            pl.BlockSpec((32, _BLKL), lambda i: (0, i)),
            pl.BlockSpec((8, _BLKL), lambda i: (0, i)),
            pl.BlockSpec((32, 1), lambda i: (0, 0)),
            pl.BlockSpec((8, 32), lambda i: (0, 0)),
            pl.BlockSpec((8, 8), lambda i: (0, 0)),
        ],
        out_specs=pl.BlockSpec((8, _BLKL), lambda i: (0, i)),
        out_shape=jax.ShapeDtypeStruct((8, _NP), jnp.float32),
    )(sT, tT, dis, bc, wfct8, bfc)


def kernel(x, c, ei, ew, W1, b1, W2, b2, Wfc, bfc):
    del c
    pad = _E_PAD - _E
    rowp = jnp.concatenate([ei[0], jnp.zeros((pad,), ei.dtype)]).reshape(_NCHUNK, _CH)
    colp = jnp.concatenate([ei[1], jnp.full((pad,), _N, ei.dtype)]).reshape(_NCHUNK, _CH)
    ewp = jnp.concatenate([ew, jnp.zeros((pad,), ew.dtype)]).reshape(_NCHUNK, _CH)
    xT = jnp.pad(x, ((0, _NP - _N), (0, 0))).T
    w1t = W1.T
    w2t = W2.T
    b1c = b1[:, None]
    b2c = b2[:, None]
    wfct8 = jnp.broadcast_to(Wfc.T, (8, 32))
    bfcb = jnp.broadcast_to(bfc.reshape(1, 1), (8, 8))

    dp = _deg_pass(colp, ewp)
    dp8 = jnp.pad(dp.reshape(2, _NP), ((0, 6), (0, 0)))
    dis, t1T = _scale_t(dp8, xT, w1t)

    def to_sc(tT):
        return jnp.transpose(tT.reshape(2, 16, _NP), (0, 2, 1)).reshape(2 * _NP, 16)

    def from_sc(s):
        return jnp.transpose(s.reshape(2, _NP, 16), (0, 2, 1)).reshape(32, _NP)

    s1 = _conv_pass(to_sc(t1T), rowp, colp, ewp)
    t2T = _finish_t(from_sc(s1), t1T, dis, b1c, w2t)
    s2 = _conv_pass(to_sc(t2T), rowp, colp, ewp)
    outT = _head_t(from_sc(s2), t2T, dis, b2c, wfct8, bfcb)
    return outT[0, :_N][:, None]


# speedup vs baseline: 39.3731x; 1.0372x over previous
import functools

import jax
import jax.numpy as jnp
from jax import lax
from jax.experimental import pallas as pl
from jax.experimental.pallas import tpu as pltpu
from jax.experimental.pallas import tpu_sc as plsc

_N = 100000
_E = 3200000
_NP = 100352
_NPT = _NP // 16
_CH = 128
_NCHUNK = 25600
_E_PAD = _NCHUNK * _CH
_CPT = _NCHUNK // 16
_SZ = 2
_NSC = _CPT // _SZ
_SZD = 8
_CPW = _NCHUNK // 32
_NSCD = _CPW // _SZD
_ESC = _SZ * _CH
_BLK = 1024
_NBLK = _NP // _BLK

_SC_PARAMS = pltpu.CompilerParams(needs_layout_passes=False,
                                  use_tc_tiling_on_sc=False)


def _deg_pass(col2, ew2):
    mesh = plsc.VectorSubcoreMesh(core_axis_name="c", subcore_axis_name="s")

    @functools.partial(
        pl.kernel,
        out_type=jax.ShapeDtypeStruct((2 * _NP,), jnp.float32),
        mesh=mesh,
        scratch_types=(
            [pltpu.VMEM_SHARED((_NP,), jnp.float32)]
            + [pltpu.VMEM((_SZD, _CH), jnp.int32) for _ in range(4)]
            + [pltpu.VMEM((_SZD, _CH), jnp.float32) for _ in range(4)]
            + [pltpu.VMEM((2048,), jnp.float32)]
            + [pltpu.SemaphoreType.DMA for _ in range(8)]
        ),
        compiler_params=_SC_PARAMS,
    )
    def k(col_hbm, ew_hbm, out_hbm, acc,
          col0, col1, col2_, col3, ew0, ew1, ew2_, ew3, zbuf,
          st0, st1, st2, st3, ss0, ss1, ss2, ss3):
        c = lax.axis_index("c")
        s = lax.axis_index("s")
        wid = c * 16 + s
        cols = (col0, col1, col2_, col3)
        ews = (ew0, ew1, ew2_, ew3)
        sst = (st0, st1, st2, st3)
        sss = (ss0, ss1, ss2, ss3)
        lo = wid * _CPW

        @pl.loop(0, 2048, step=16)
        def _(i):
            zbuf[pl.ds(i, 16)] = jnp.zeros((16,), jnp.float32)

        @pl.loop(0, 3)
        def _(i):
            pltpu.sync_copy(zbuf, acc.at[pl.ds(s * _NPT + i * 2048, 2048)])

        pltpu.sync_copy(zbuf.at[pl.ds(0, _NPT - 3 * 2048)],
                        acc.at[pl.ds(s * _NPT + 3 * 2048, _NPT - 3 * 2048)])
        plsc.subcore_barrier()

        def stage(b, n):
            base = lo + n * _SZD
            pltpu.async_copy(col_hbm.at[pl.ds(base, _SZD)], cols[b], sst[b])
            pltpu.async_copy(ew_hbm.at[pl.ds(base, _SZD)], ews[b], sst[b])

        def wait_stage(b):
            pltpu.make_async_copy(col_hbm.at[pl.ds(0, _SZD)], cols[b], sst[b]).wait()
            pltpu.make_async_copy(ew_hbm.at[pl.ds(0, _SZD)], ews[b], sst[b]).wait()

        def fire_scatters(b):
            for kk in range(_SZD):
                pltpu.async_copy(ews[b].at[kk], acc.at[cols[b].at[kk]],
                                 sss[b], add=True)

        def drain_scatters(b):
            pltpu.make_async_copy(ew_hbm.at[pl.ds(0, _SZD)], ews[b], sss[b]).wait()

        stage(0, 0)
        stage(1, 1)

        @pl.loop(0, _NSCD // 4)
        def _(q):
            for j in range(4):
                n = q * 4 + j
                bn = j
                b2 = (j + 2) % 4

                def prep(n=n, bn=bn, b2=b2):
                    drain_scatters(b2)

                if j < 2:
                    @pl.when(q > 0)
                    def _():
                        prep()
                else:
                    prep()

                def st(n=n, b2=b2):
                    stage(b2, n + 2)

                if j < 2:
                    st()
                else:
                    @pl.when(q < _NSCD // 4 - 1)
                    def _():
                        st()

                wait_stage(bn)
                fire_scatters(bn)

        drain_scatters((_NSCD - 2) % 4)
        drain_scatters((_NSCD - 1) % 4)
        plsc.subcore_barrier()
        pltpu.sync_copy(
            acc.at[pl.ds(s * _NPT, _NPT)],
            out_hbm.at[pl.ds(c * _NP + s * _NPT, _NPT)],
        )

    return k(col2, ew2)


def _conv_pass(tbl, row2, col2, ew2):
    mesh = plsc.VectorSubcoreMesh(core_axis_name="c", subcore_axis_name="s")

    @functools.partial(
        pl.kernel,
        out_type=jax.ShapeDtypeStruct((2 * _NP, 16), jnp.float32),
        mesh=mesh,
        scratch_types=(
            [pltpu.VMEM_SHARED((_NP, 16), jnp.float32)]
            + [pltpu.VMEM((_SZ, _CH), jnp.int32) for _ in range(8)]
            + [pltpu.VMEM((_SZ, _CH), jnp.float32) for _ in range(4)]
            + [pltpu.VMEM((_ESC, 16), jnp.float32) for _ in range(4)]
            + [pltpu.VMEM((_CH, 16), jnp.float32)]
            + [pltpu.SemaphoreType.DMA for _ in range(12)]
        ),
        compiler_params=_SC_PARAMS,
    )
    def k(t_hbm, row_hbm, col_hbm, ew_hbm, out_hbm, acc,
          row0, row1, row2_, row3, col0, col1, col2_, col3,
          ew0, ew1, ew2_, ew3, msg0, msg1, msg2, msg3, zbuf,
          st0, st1, st2, st3, sg0, sg1, sg2, sg3, ss0, ss1, ss2, ss3):
        c = lax.axis_index("c")
        s = lax.axis_index("s")
        cn = c * _NP
        rows = (row0, row1, row2_, row3)
        cols = (col0, col1, col2_, col3)
        ews = (ew0, ew1, ew2_, ew3)
        msgs = (msg0, msg1, msg2, msg3)
        sst = (st0, st1, st2, st3)
        ssg = (sg0, sg1, sg2, sg3)
        sss = (ss0, ss1, ss2, ss3)
        lo = s * _CPT

        @pl.loop(0, _CH)
        def _(i):
            zbuf[i] = jnp.zeros((16,), jnp.float32)

        @pl.loop(0, _NPT, step=_CH)
        def _(i):
            pltpu.sync_copy(zbuf, acc.at[pl.ds(s * _NPT + i, _CH)])

        plsc.subcore_barrier()

        iota = lax.iota(jnp.int32, 16)
        dconsts = [jnp.full((16,), d, jnp.int32) for d in range(16)]

        def stage(b, n):
            base = lo + n * _SZ
            pltpu.async_copy(row_hbm.at[pl.ds(base, _SZ)], rows[b], sst[b])
            pltpu.async_copy(col_hbm.at[pl.ds(base, _SZ)], cols[b], sst[b])
            pltpu.async_copy(ew_hbm.at[pl.ds(base, _SZ)], ews[b], sst[b])

        def wait_stage(b):
            pltpu.make_async_copy(row_hbm.at[pl.ds(0, _SZ)], rows[b], sst[b]).wait()
            pltpu.make_async_copy(col_hbm.at[pl.ds(0, _SZ)], cols[b], sst[b]).wait()
            pltpu.make_async_copy(ew_hbm.at[pl.ds(0, _SZ)], ews[b], sst[b]).wait()

        def adjust(b):
            rowX = rows[b]

            @pl.loop(0, _SZ)
            def _(kk):
                for i in range(8):
                    rowX[kk, pl.ds(i * 16, 16)] = rowX[kk, pl.ds(i * 16, 16)] + cn

        def fire_gathers(b):
            for kk in range(_SZ):
                pltpu.async_copy(t_hbm.at[rows[b].at[kk]],
                                 msgs[b].at[pl.ds(kk * _CH, _CH)], ssg[b])

        def wait_gathers(b):
            pltpu.make_async_copy(t_hbm.at[pl.ds(0, _ESC)], msgs[b], ssg[b]).wait()

        def multiply(b):
            ewX, msgX = ews[b], msgs[b]

            @pl.loop(0, _ESC, step=16)
            def _(e0):
                kk = e0 // _CH
                off = e0 - kk * _CH
                ew_v = ewX[kk, pl.ds(off, 16)]
                for l in range(16):
                    bc = lax.gather(
                        ew_v, dconsts[l][:, None],
                        lax.GatherDimensionNumbers(
                            offset_dims=(), collapsed_slice_dims=(0,),
                            start_index_map=(0,)),
                        slice_sizes=(1,),
                        mode=lax.GatherScatterMode.PROMISE_IN_BOUNDS)
                    msgX[e0 + l] = msgX[e0 + l] * bc

        def fire_scatters(b):
            for kk in range(_SZ):
                pltpu.async_copy(msgs[b].at[pl.ds(kk * _CH, _CH)],
                                 acc.at[cols[b].at[kk]], sss[b], add=True)

        def drain_scatters(b):
            pltpu.make_async_copy(t_hbm.at[pl.ds(0, _ESC)], msgs[b], sss[b]).wait()

        stage(0, 0)
        stage(1, 1)
        wait_stage(0)
        adjust(0)
        fire_gathers(0)

        @pl.loop(0, _NSC // 4)
        def _(q):
            for j in range(4):
                n = q * 4 + j
                bn = j
                b1 = (j + 1) % 4
                b2 = (j + 2) % 4

                def drain2(b2=b2):
                    drain_scatters(b2)

                if j < 2:
                    @pl.when(q > 0)
                    def _():
                        drain2()
                else:
                    drain2()

                def st2(n=n, b2=b2):
                    stage(b2, n + 2)

                if j < 2:
                    st2()
                else:
                    @pl.when(q < _NSC // 4 - 1)
                    def _():
                        st2()

                def launch1(b1=b1):
                    wait_stage(b1)
                    adjust(b1)
                    fire_gathers(b1)

                if j < 3:
                    launch1()
                else:
                    @pl.when(q < _NSC // 4 - 1)
                    def _():
                        launch1()

                wait_gathers(bn)
                multiply(bn)
                fire_scatters(bn)

        drain_scatters((_NSC - 2) % 4)
        drain_scatters((_NSC - 1) % 4)
        plsc.subcore_barrier()
        pltpu.sync_copy(
            acc.at[pl.ds(s * _NPT, _NPT)],
            out_hbm.at[pl.ds(cn + s * _NPT, _NPT)],
        )

    return k(tbl, row2, col2, ew2)


_NBF = 7
_HROWS = _NP * 16 // 128
_DROWS = _NP // 128
_BLKF = _HROWS // _NBF


def _dis_kernel(dp0_ref, dp1_ref, dis_ref):
    dis_ref[...] = lax.rsqrt(dp0_ref[...] + dp1_ref[...] + 1.0)


def _dis_flat(dp0v, dp1v):
    return pl.pallas_call(
        _dis_kernel,
        out_shape=jax.ShapeDtypeStruct((_DROWS, 128), jnp.float32),
    )(dp0v, dp1v)


def _first_kernel(de_ref, xf_ref, w_ref, t_ref):
    t_ref[...] = de_ref[...] * jnp.dot(xf_ref[...], w_ref[0],
                                       preferred_element_type=jnp.float32)


def _first_flat(de, xflat, wa1):
    return pl.pallas_call(
        _first_kernel,
        grid=(2, _NBF),
        in_specs=[
            pl.BlockSpec((_BLKF, 128), lambda h, i: (i, 0)),
            pl.BlockSpec((_BLKF, 128), lambda h, i: (i, 0)),
            pl.BlockSpec((1, 128, 128), lambda h, i: (h, 0, 0)),
        ],
        out_specs=pl.BlockSpec((_BLKF, 128), lambda h, i: (h * _NBF + i, 0)),
        out_shape=jax.ShapeDtypeStruct((2 * _HROWS, 128), jnp.float32),
    )(de, xflat, wa1)


def _layer_kernel(s0_ref, s1_ref, t0_ref, t1_ref, de_ref, b_ref, wa_ref,
                  wb_ref, o_ref):
    de = de_ref[...]
    h0 = jax.nn.relu(de * (s0_ref[...] + t0_ref[...]) + b_ref[0:1, :])
    h1 = jax.nn.relu(de * (s1_ref[...] + t1_ref[...]) + b_ref[1:2, :])
    xw = (jnp.dot(h0, wa_ref[0], preferred_element_type=jnp.float32)
          + jnp.dot(h1, wb_ref[0], preferred_element_type=jnp.float32))
    o_ref[...] = de * xw


def _layer_flat(sflat, tflat, de, b2r, wa, wb):
    return pl.pallas_call(
        _layer_kernel,
        grid=(2, _NBF),
        in_specs=[
            pl.BlockSpec((_BLKF, 128), lambda h, i: (i, 0)),
            pl.BlockSpec((_BLKF, 128), lambda h, i: (_NBF + i, 0)),
            pl.BlockSpec((_BLKF, 128), lambda h, i: (i, 0)),
            pl.BlockSpec((_BLKF, 128), lambda h, i: (_NBF + i, 0)),
            pl.BlockSpec((_BLKF, 128), lambda h, i: (i, 0)),
            pl.BlockSpec((2, 128), lambda h, i: (0, 0)),
            pl.BlockSpec((1, 128, 128), lambda h, i: (h, 0, 0)),
            pl.BlockSpec((1, 128, 128), lambda h, i: (h, 0, 0)),
        ],
        out_specs=pl.BlockSpec((_BLKF, 128), lambda h, i: (h * _NBF + i, 0)),
        out_shape=jax.ShapeDtypeStruct((2 * _HROWS, 128), jnp.float32),
    )(sflat, sflat, tflat, tflat, de, b2r, wa, wb)


def _head_kernel(s0_ref, s1_ref, t0_ref, t1_ref, de_ref, b_ref, c0_ref,
                 c1_ref, bfc_ref, o_ref):
    de = de_ref[...]
    h0 = jax.nn.relu(de * (s0_ref[...] + t0_ref[...]) + b_ref[0:1, :])
    h1 = jax.nn.relu(de * (s1_ref[...] + t1_ref[...]) + b_ref[1:2, :])
    o = (jnp.dot(h0, c0_ref[...], preferred_element_type=jnp.float32)
         + jnp.dot(h1, c1_ref[...], preferred_element_type=jnp.float32))
    o_ref[...] = o + bfc_ref[0, 0]


def _head_flat(sflat, tflat, de, b2r, c0, c1, bfc):
    return pl.pallas_call(
        _head_kernel,
        grid=(_NBF,),
        in_specs=[
            pl.BlockSpec((_BLKF, 128), lambda i: (i, 0)),
            pl.BlockSpec((_BLKF, 128), lambda i: (_NBF + i, 0)),
            pl.BlockSpec((_BLKF, 128), lambda i: (i, 0)),
            pl.BlockSpec((_BLKF, 128), lambda i: (_NBF + i, 0)),
            pl.BlockSpec((_BLKF, 128), lambda i: (i, 0)),
            pl.BlockSpec((2, 128), lambda i: (0, 0)),
            pl.BlockSpec((128, 128), lambda i: (0, 0)),
            pl.BlockSpec((128, 128), lambda i: (0, 0)),
            pl.BlockSpec((8, 8), lambda i: (0, 0)),
        ],
        out_specs=pl.BlockSpec((_BLKF, 128), lambda i: (i, 0)),
        out_shape=jax.ShapeDtypeStruct((_HROWS, 128), jnp.float32),
    )(sflat, sflat, tflat, tflat, de, b2r, c0, c1, bfc)


def kernel(x, c, ei, ew, W1, b1, W2, b2, Wfc, bfc):
    del c
    pad = _E_PAD - _E
    rowp = jnp.concatenate([ei[0], jnp.zeros((pad,), ei.dtype)]).reshape(_NCHUNK, _CH)
    colp = jnp.concatenate([ei[1], jnp.full((pad,), _N, ei.dtype)]).reshape(_NCHUNK, _CH)
    ewp = jnp.concatenate([ew, jnp.zeros((pad,), ew.dtype)]).reshape(_NCHUNK, _CH)
    xflat = jnp.pad(x, ((0, _NP - _N), (0, 0))).reshape(_HROWS, 128)
    b1r = jnp.concatenate([jnp.tile(b1[:16], 8), jnp.tile(b1[16:], 8)]).reshape(2, 128)
    b2r = jnp.concatenate([jnp.tile(b2[:16], 8), jnp.tile(b2[16:], 8)]).reshape(2, 128)
    bfcb = jnp.broadcast_to(bfc.reshape(1, 1), (8, 8))
    eye8 = jnp.eye(8, dtype=jnp.float32)

    def bd(w16):
        return jnp.kron(eye8, w16)

    wa1 = jnp.stack([bd(W1[:, :16]), bd(W1[:, 16:])])
    wa2 = jnp.stack([bd(W2[:16, :16]), bd(W2[:16, 16:])])
    wb2 = jnp.stack([bd(W2[16:, :16]), bd(W2[16:, 16:])])
    c0 = bd(jnp.tile(Wfc[:16], (1, 16)))
    c1 = bd(jnp.tile(Wfc[16:], (1, 16)))

    dp = _deg_pass(colp, ewp)
    dis = _dis_flat(dp[:_NP].reshape(_DROWS, 128),
                    dp[_NP:].reshape(_DROWS, 128))
    de = jnp.repeat(dis.reshape(_NP), 16).reshape(_HROWS, 128)
    t1f = _first_flat(de, xflat, wa1)
    s1 = _conv_pass(t1f.reshape(2 * _NP, 16), rowp, colp, ewp)
    t2f = _layer_flat(s1.reshape(2 * _HROWS, 128), t1f, de, b1r, wa2, wb2)
    s2 = _conv_pass(t2f.reshape(2 * _NP, 16), rowp, colp, ewp)
    outf = _head_flat(s2.reshape(2 * _HROWS, 128), t2f, de, b2r, c0, c1, bfcb)
    return outf.reshape(_NP, 16)[:_N, :1]
